# baseline (speedup 1.0000x reference)
import functools

import jax
import jax.numpy as jnp
from jax import lax
from jax.experimental import pallas as pl
from jax.experimental.pallas import tpu as pltpu
from jax.experimental.pallas import tpu_sc as plsc

B, H, W = 32, 512, 512
P = H * W
NUM_CLASSES = 4
C = 8192
NCHUNK = P // C
LANES = 16

_mesh = plsc.VectorSubcoreMesh(core_axis_name="c", subcore_axis_name="s")


@functools.partial(
    pl.kernel,
    out_type=jax.ShapeDtypeStruct((B, NUM_CLASSES, P), jnp.float32),
    mesh=_mesh,
    scratch_types=[
        pltpu.VMEM((C,), jnp.float32),
        pltpu.VMEM((C,), jnp.float32),
        pltpu.VMEM((NUM_CLASSES, C), jnp.float32),
        pltpu.VMEM((NUM_CLASSES, C), jnp.float32),
        pltpu.SemaphoreType.DMA,
        pltpu.SemaphoreType.DMA,
        pltpu.SemaphoreType.DMA,
        pltpu.SemaphoreType.DMA,
    ],
)
def _onehot_sc(x_hbm, out_hbm, x_v0, x_v1, o_v0, o_v1,
               si0, si1, so0, so1):
    num_cores = 2
    b = lax.axis_index("s") * num_cores + lax.axis_index("c")
    x_bufs = (x_v0, x_v1)
    o_bufs = (o_v0, o_v1)
    in_sems = (si0, si1)
    out_sems = (so0, so1)

    def in_src(j):
        return x_hbm.at[b, pl.ds(j * C, C)]

    def out_dst(j, cls):
        return out_hbm.at[b, cls, pl.ds(j * C, C)]

    pltpu.async_copy(in_src(0), x_bufs[0], in_sems[0])
    pltpu.async_copy(in_src(1), x_bufs[1], in_sems[1])

    def pair_body(i, carry):
        for t in range(2):
            j = i * 2 + t
            x_v, o_v = x_bufs[t], o_bufs[t]
            pltpu.make_async_copy(in_src(j), x_v, in_sems[t]).wait()

            @pl.when(j >= 2)
            def _():
                for cls in range(NUM_CLASSES):
                    pltpu.make_async_copy(
                        o_v.at[cls], out_dst(j - 2, cls), out_sems[t]).wait()

            @plsc.parallel_loop(0, C, step=LANES, unroll=16)
            def _vec(k):
                sl = pl.ds(k, LANES)
                v = x_v[sl]
                one = jnp.ones((LANES,), jnp.float32)
                zero = jnp.zeros((LANES,), jnp.float32)
                s0 = jnp.where(v < 0.1, one, zero)
                s1 = jnp.where(v < 1.0, one, zero)
                s2 = jnp.where(v < 2.5, one, zero)
                o_v[0, sl] = s0
                o_v[1, sl] = s1 - s0
                o_v[2, sl] = s2 - s1
                o_v[3, sl] = one - s2

            for cls in range(NUM_CLASSES):
                pltpu.async_copy(o_v.at[cls], out_dst(j, cls), out_sems[t])

            @pl.when(j + 2 < NCHUNK)
            def _():
                pltpu.async_copy(in_src(j + 2), x_v, in_sems[t])
        return carry

    lax.fori_loop(0, NCHUNK // 2, pair_body, 0)

    for t in range(2):
        j = NCHUNK - 2 + t
        for cls in range(NUM_CLASSES):
            pltpu.make_async_copy(
                o_bufs[t].at[cls], out_dst(j, cls), out_sems[t]).wait()


def kernel(x):
    x2d = x.reshape(B, P)
    out = _onehot_sc(x2d)
    return out.reshape(B, 1, NUM_CLASSES, H, W)


# speedup vs baseline: 1.1120x; 1.1120x over previous
import functools

import jax
import jax.numpy as jnp
from jax import lax
from jax.experimental import pallas as pl
from jax.experimental.pallas import tpu as pltpu
from jax.experimental.pallas import tpu_sc as plsc

B, H, W = 32, 512, 512
P = H * W
NUM_CLASSES = 4
C = 8192
NCHUNK = P // C
LANES = 16

_mesh = plsc.VectorSubcoreMesh(core_axis_name="c", subcore_axis_name="s")


@functools.partial(
    pl.kernel,
    out_type=jax.ShapeDtypeStruct((B, NUM_CLASSES, P), jnp.float32),
    mesh=_mesh,
    scratch_types=[
        pltpu.VMEM((C,), jnp.float32),
        pltpu.VMEM((C,), jnp.float32),
        pltpu.VMEM((NUM_CLASSES, C), jnp.float32),
        pltpu.VMEM((NUM_CLASSES, C), jnp.float32),
        pltpu.SemaphoreType.DMA,
        pltpu.SemaphoreType.DMA,
        pltpu.SemaphoreType.DMA,
        pltpu.SemaphoreType.DMA,
    ],
)
def _onehot_sc(x_hbm, out_hbm, x_v0, x_v1, o_v0, o_v1,
               si0, si1, so0, so1):
    num_cores = 2
    b = lax.axis_index("s") * num_cores + lax.axis_index("c")
    x_bufs = (x_v0, x_v1)
    o_bufs = (o_v0, o_v1)
    in_sems = (si0, si1)
    out_sems = (so0, so1)

    def in_src(j):
        return x_hbm.at[b, pl.ds(j * C, C)]

    def out_dst(j):
        return out_hbm.at[b, :, pl.ds(j * C, C)]

    pltpu.async_copy(in_src(0), x_bufs[0], in_sems[0])
    pltpu.async_copy(in_src(1), x_bufs[1], in_sems[1])

    def pair_body(i, carry):
        for t in range(2):
            j = i * 2 + t
            x_v, o_v = x_bufs[t], o_bufs[t]
            pltpu.make_async_copy(in_src(j), x_v, in_sems[t]).wait()

            @pl.when(j >= 2)
            def _():
                pltpu.make_async_copy(o_v, out_dst(j - 2), out_sems[t]).wait()

            @plsc.parallel_loop(0, C, step=LANES, unroll=8)
            def _vec(k):
                sl = pl.ds(k, LANES)
                v = x_v[sl]
                one = jnp.ones((LANES,), jnp.float32)
                zero = jnp.zeros((LANES,), jnp.float32)
                s0 = jnp.where(v < 0.1, one, zero)
                s1 = jnp.where(v < 1.0, one, zero)
                s2 = jnp.where(v < 2.5, one, zero)
                o_v[0, sl] = s0
                o_v[1, sl] = s1 - s0
                o_v[2, sl] = s2 - s1
                o_v[3, sl] = one - s2

            pltpu.async_copy(o_v, out_dst(j), out_sems[t])

            @pl.when(j + 2 < NCHUNK)
            def _():
                pltpu.async_copy(in_src(j + 2), x_v, in_sems[t])
        return carry

    lax.fori_loop(0, NCHUNK // 2, pair_body, 0)

    for t in range(2):
        j = NCHUNK - 2 + t
        pltpu.make_async_copy(o_bufs[t], out_dst(j), out_sems[t]).wait()


def kernel(x):
    x2d = x.reshape(B, P)
    out = _onehot_sc(x2d)
    return out.reshape(B, 1, NUM_CLASSES, H, W)


# speedup vs baseline: 1.1133x; 1.0012x over previous
import functools

import jax
import jax.numpy as jnp
from jax import lax
from jax.experimental import pallas as pl
from jax.experimental.pallas import tpu as pltpu
from jax.experimental.pallas import tpu_sc as plsc

B, H, W = 32, 512, 512
P = H * W
NUM_CLASSES = 4
C = 4096
NCHUNK = P // C
NBUF = 4
LANES = 16

_mesh = plsc.VectorSubcoreMesh(core_axis_name="c", subcore_axis_name="s")


@functools.partial(
    pl.kernel,
    out_type=jax.ShapeDtypeStruct((B, NUM_CLASSES, P), jnp.float32),
    mesh=_mesh,
    scratch_types=(
        [pltpu.VMEM((C,), jnp.float32) for _ in range(NBUF)]
        + [pltpu.VMEM((NUM_CLASSES, C), jnp.float32) for _ in range(NBUF)]
        + [pltpu.SemaphoreType.DMA for _ in range(2 * NBUF)]
    ),
)
def _onehot_sc(x_hbm, out_hbm, *scratch):
    x_bufs = scratch[:NBUF]
    o_bufs = scratch[NBUF:2 * NBUF]
    in_sems = scratch[2 * NBUF:3 * NBUF]
    out_sems = scratch[3 * NBUF:4 * NBUF]
    num_cores = 2
    b = lax.axis_index("s") * num_cores + lax.axis_index("c")

    def in_src(j):
        return x_hbm.at[b, pl.ds(j * C, C)]

    def out_dst(j):
        return out_hbm.at[b, :, pl.ds(j * C, C)]

    for t in range(NBUF):
        pltpu.async_copy(in_src(t), x_bufs[t], in_sems[t])

    def ring_body(i, carry):
        for t in range(NBUF):
            j = i * NBUF + t
            x_v, o_v = x_bufs[t], o_bufs[t]
            pltpu.make_async_copy(in_src(j), x_v, in_sems[t]).wait()

            @pl.when(j >= NBUF)
            def _():
                pltpu.make_async_copy(
                    o_v, out_dst(j - NBUF), out_sems[t]).wait()

            @plsc.parallel_loop(0, C, step=LANES, unroll=8)
            def _vec(k):
                sl = pl.ds(k, LANES)
                v = x_v[sl]
                one = jnp.ones((LANES,), jnp.float32)
                zero = jnp.zeros((LANES,), jnp.float32)
                s0 = jnp.where(v < 0.1, one, zero)
                s1 = jnp.where(v < 1.0, one, zero)
                s2 = jnp.where(v < 2.5, one, zero)
                o_v[0, sl] = s0
                o_v[1, sl] = s1 - s0
                o_v[2, sl] = s2 - s1
                o_v[3, sl] = one - s2

            pltpu.async_copy(o_v, out_dst(j), out_sems[t])

            @pl.when(j + NBUF < NCHUNK)
            def _():
                pltpu.async_copy(in_src(j + NBUF), x_v, in_sems[t])
        return carry

    lax.fori_loop(0, NCHUNK // NBUF, ring_body, 0)

    for t in range(NBUF):
        j = NCHUNK - NBUF + t
        pltpu.make_async_copy(o_bufs[t], out_dst(j), out_sems[t]).wait()


def kernel(x):
    x2d = x.reshape(B, P)
    out = _onehot_sc(x2d)
    return out.reshape(B, 1, NUM_CLASSES, H, W)


# speedup vs baseline: 1.1197x; 1.0057x over previous
import functools

import jax
import jax.numpy as jnp
from jax import lax
from jax.experimental import pallas as pl
from jax.experimental.pallas import tpu as pltpu
from jax.experimental.pallas import tpu_sc as plsc

B, H, W = 32, 512, 512
P = H * W
NUM_CLASSES = 4
C = 8192
NBUF = 2
LANES = 16
K_SC = 8
NTILE = 32
SPAN = P * K_SC // NTILE
NCHUNK = SPAN // C

_mesh = plsc.VectorSubcoreMesh(core_axis_name="c", subcore_axis_name="s")


@functools.partial(
    pl.kernel,
    out_type=jax.ShapeDtypeStruct((B, NUM_CLASSES, P), jnp.float32),
    mesh=_mesh,
    scratch_types=(
        [pltpu.VMEM((C,), jnp.float32) for _ in range(NBUF)]
        + [pltpu.VMEM((NUM_CLASSES, C), jnp.float32) for _ in range(NBUF)]
        + [pltpu.SemaphoreType.DMA for _ in range(2 * NBUF)]
    ),
)
def _onehot_sc(x_hbm, out_hbm, *scratch):
    x_bufs = scratch[:NBUF]
    o_bufs = scratch[NBUF:2 * NBUF]
    in_sems = scratch[2 * NBUF:3 * NBUF]
    out_sems = scratch[3 * NBUF:4 * NBUF]
    num_cores = 2
    w = lax.axis_index("s") * num_cores + lax.axis_index("c")
    tiles_per_row = NTILE // K_SC
    b = w // tiles_per_row
    base = (w % tiles_per_row) * SPAN

    def in_src(j):
        return x_hbm.at[b, pl.ds(base + j * C, C)]

    def out_dst(j):
        return out_hbm.at[b, :, pl.ds(base + j * C, C)]

    for t in range(NBUF):
        pltpu.async_copy(in_src(t), x_bufs[t], in_sems[t])

    def ring_body(i, carry):
        for t in range(NBUF):
            j = i * NBUF + t
            x_v, o_v = x_bufs[t], o_bufs[t]
            pltpu.make_async_copy(in_src(j), x_v, in_sems[t]).wait()

            @pl.when(j >= NBUF)
            def _():
                pltpu.make_async_copy(
                    o_v, out_dst(j - NBUF), out_sems[t]).wait()

            @plsc.parallel_loop(0, C, step=LANES, unroll=8)
            def _vec(k):
                sl = pl.ds(k, LANES)
                v = x_v[sl]
                one = jnp.ones((LANES,), jnp.float32)
                zero = jnp.zeros((LANES,), jnp.float32)
                s0 = jnp.where(v < 0.1, one, zero)
                s1 = jnp.where(v < 1.0, one, zero)
                s2 = jnp.where(v < 2.5, one, zero)
                o_v[0, sl] = s0
                o_v[1, sl] = s1 - s0
                o_v[2, sl] = s2 - s1
                o_v[3, sl] = one - s2

            pltpu.async_copy(o_v, out_dst(j), out_sems[t])

            @pl.when(j + NBUF < NCHUNK)
            def _():
                pltpu.async_copy(in_src(j + NBUF), x_v, in_sems[t])
        return carry

    lax.fori_loop(0, NCHUNK // NBUF, ring_body, 0)

    for t in range(NBUF):
        j = NCHUNK - NBUF + t
        pltpu.make_async_copy(o_bufs[t], out_dst(j), out_sems[t]).wait()


def _tc_body(x_ref, buf_ref, o_ref):
    del buf_ref
    v = x_ref[0]
    one = jnp.ones_like(v)
    zero = jnp.zeros_like(v)
    s0 = jnp.where(v < 0.1, one, zero)
    s1 = jnp.where(v < 1.0, one, zero)
    s2 = jnp.where(v < 2.5, one, zero)
    o_ref[0, 0] = s0
    o_ref[0, 1] = s1 - s0
    o_ref[0, 2] = s2 - s1
    o_ref[0, 3] = one - s2


_tc_call = pl.pallas_call(
    _tc_body,
    grid=(B - K_SC,),
    in_specs=[
        pl.BlockSpec((1, H, W), lambda i: (i + K_SC, 0, 0)),
        pl.BlockSpec(memory_space=pltpu.MemorySpace.HBM),
    ],
    out_specs=pl.BlockSpec((1, NUM_CLASSES, H, W), lambda i: (i + K_SC, 0, 0, 0)),
    out_shape=jax.ShapeDtypeStruct((B, NUM_CLASSES, H, W), jnp.float32),
    input_output_aliases={1: 0},
)


def kernel(x):
    x2d = x.reshape(B, P)
    buf = _onehot_sc(x2d)
    out = _tc_call(x.reshape(B, H, W), buf.reshape(B, NUM_CLASSES, H, W))
    return out.reshape(B, 1, NUM_CLASSES, H, W)


# speedup vs baseline: 4.4513x; 3.9755x over previous
import functools

import jax
import jax.numpy as jnp
from jax import lax
from jax.experimental import pallas as pl
from jax.experimental.pallas import tpu as pltpu
from jax.experimental.pallas import tpu_sc as plsc

B, H, W = 32, 512, 512
NUM_CLASSES = 4
R = 16
NCHUNK = H // R
NBUF = 2
LANES = 16

_mesh = plsc.VectorSubcoreMesh(core_axis_name="c", subcore_axis_name="s")


@functools.partial(
    pl.kernel,
    out_type=jax.ShapeDtypeStruct((B, NUM_CLASSES, H, W), jnp.float32),
    mesh=_mesh,
    compiler_params=pltpu.CompilerParams(use_tc_tiling_on_sc=True),
    scratch_types=(
        [pltpu.VMEM((R, W), jnp.float32) for _ in range(NBUF)]
        + [pltpu.VMEM((NUM_CLASSES, R, W), jnp.float32) for _ in range(NBUF)]
        + [pltpu.SemaphoreType.DMA for _ in range(2 * NBUF)]
    ),
)
def _onehot_sc(x_hbm, out_hbm, *scratch):
    x_bufs = scratch[:NBUF]
    o_bufs = scratch[NBUF:2 * NBUF]
    in_sems = scratch[2 * NBUF:3 * NBUF]
    out_sems = scratch[3 * NBUF:4 * NBUF]
    num_cores = 2
    b = lax.axis_index("s") * num_cores + lax.axis_index("c")

    def in_src(j):
        return x_hbm.at[b, pl.ds(j * R, R), :]

    def out_dst(j):
        return out_hbm.at[b, :, pl.ds(j * R, R), :]

    for t in range(NBUF):
        pltpu.async_copy(in_src(t), x_bufs[t], in_sems[t])

    def ring_body(i, carry):
        for t in range(NBUF):
            j = i * NBUF + t
            x_v, o_v = x_bufs[t], o_bufs[t]
            pltpu.make_async_copy(in_src(j), x_v, in_sems[t]).wait()

            @pl.when(j >= NBUF)
            def _():
                pltpu.make_async_copy(
                    o_v, out_dst(j - NBUF), out_sems[t]).wait()

            @plsc.parallel_loop(0, W, step=LANES)
            def _vec(k):
                sl = pl.ds(k, LANES)
                one = jnp.ones((LANES,), jnp.float32)
                zero = jnp.zeros((LANES,), jnp.float32)
                for r in range(R):
                    v = x_v[r, sl]
                    s0 = jnp.where(v < 0.1, one, zero)
                    s1 = jnp.where(v < 1.0, one, zero)
                    s2 = jnp.where(v < 2.5, one, zero)
                    o_v[0, r, sl] = s0
                    o_v[1, r, sl] = s1 - s0
                    o_v[2, r, sl] = s2 - s1
                    o_v[3, r, sl] = one - s2

            pltpu.async_copy(o_v, out_dst(j), out_sems[t])

            @pl.when(j + NBUF < NCHUNK)
            def _():
                pltpu.async_copy(in_src(j + NBUF), x_v, in_sems[t])
        return carry

    lax.fori_loop(0, NCHUNK // NBUF, ring_body, 0)

    for t in range(NBUF):
        j = NCHUNK - NBUF + t
        pltpu.make_async_copy(o_bufs[t], out_dst(j), out_sems[t]).wait()


def kernel(x):
    out = _onehot_sc(x.reshape(B, H, W))
    return out.reshape(B, 1, NUM_CLASSES, H, W)


# speedup vs baseline: 4.5406x; 1.0201x over previous
import functools

import jax
import jax.numpy as jnp
from jax import lax
from jax.experimental import pallas as pl
from jax.experimental.pallas import tpu as pltpu
from jax.experimental.pallas import tpu_sc as plsc

B, H, W = 32, 512, 512
NUM_CLASSES = 4
R = 16
NBUF = 2
LANES = 16
K_SC = 8
NTILE = 32
TPB = NTILE // K_SC
SPAN = H // TPB
NCHUNK = SPAN // R

_mesh = plsc.VectorSubcoreMesh(core_axis_name="c", subcore_axis_name="s")


@functools.partial(
    pl.kernel,
    out_type=jax.ShapeDtypeStruct((B, NUM_CLASSES, H, W), jnp.float32),
    mesh=_mesh,
    compiler_params=pltpu.CompilerParams(use_tc_tiling_on_sc=True),
    scratch_types=(
        [pltpu.VMEM((R, W), jnp.float32) for _ in range(NBUF)]
        + [pltpu.VMEM((NUM_CLASSES, R, W), jnp.float32) for _ in range(NBUF)]
        + [pltpu.SemaphoreType.DMA for _ in range(2 * NBUF)]
    ),
)
def _onehot_sc(x_hbm, out_hbm, *scratch):
    x_bufs = scratch[:NBUF]
    o_bufs = scratch[NBUF:2 * NBUF]
    in_sems = scratch[2 * NBUF:3 * NBUF]
    out_sems = scratch[3 * NBUF:4 * NBUF]
    num_cores = 2
    w = lax.axis_index("s") * num_cores + lax.axis_index("c")
    b = w // TPB
    base = (w % TPB) * SPAN

    def in_src(j):
        return x_hbm.at[b, pl.ds(base + j * R, R), :]

    def out_dst(j):
        return out_hbm.at[b, :, pl.ds(base + j * R, R), :]

    for t in range(NBUF):
        pltpu.async_copy(in_src(t), x_bufs[t], in_sems[t])

    def ring_body(i, carry):
        for t in range(NBUF):
            j = i * NBUF + t
            x_v, o_v = x_bufs[t], o_bufs[t]
            pltpu.make_async_copy(in_src(j), x_v, in_sems[t]).wait()

            @pl.when(j >= NBUF)
            def _():
                pltpu.make_async_copy(
                    o_v, out_dst(j - NBUF), out_sems[t]).wait()

            @plsc.parallel_loop(0, W, step=LANES)
            def _vec(k):
                sl = pl.ds(k, LANES)
                one = jnp.ones((LANES,), jnp.float32)
                zero = jnp.zeros((LANES,), jnp.float32)
                for r in range(R):
                    v = x_v[r, sl]
                    s0 = jnp.where(v < 0.1, one, zero)
                    s1 = jnp.where(v < 1.0, one, zero)
                    s2 = jnp.where(v < 2.5, one, zero)
                    o_v[0, r, sl] = s0
                    o_v[1, r, sl] = s1 - s0
                    o_v[2, r, sl] = s2 - s1
                    o_v[3, r, sl] = one - s2

            pltpu.async_copy(o_v, out_dst(j), out_sems[t])

            @pl.when(j + NBUF < NCHUNK)
            def _():
                pltpu.async_copy(in_src(j + NBUF), x_v, in_sems[t])
        return carry

    lax.fori_loop(0, NCHUNK // NBUF, ring_body, 0)

    for t in range(NBUF):
        j = NCHUNK - NBUF + t
        pltpu.make_async_copy(o_bufs[t], out_dst(j), out_sems[t]).wait()


def _tc_body(x_ref, buf_ref, o_ref):
    del buf_ref
    v = x_ref[0]
    one = jnp.ones_like(v)
    zero = jnp.zeros_like(v)
    s0 = jnp.where(v < 0.1, one, zero)
    s1 = jnp.where(v < 1.0, one, zero)
    s2 = jnp.where(v < 2.5, one, zero)
    o_ref[0, 0] = s0
    o_ref[0, 1] = s1 - s0
    o_ref[0, 2] = s2 - s1
    o_ref[0, 3] = one - s2


_tc_call = pl.pallas_call(
    _tc_body,
    grid=(B - K_SC,),
    in_specs=[
        pl.BlockSpec((1, H, W), lambda i: (i + K_SC, 0, 0)),
        pl.BlockSpec(memory_space=pltpu.MemorySpace.HBM),
    ],
    out_specs=pl.BlockSpec((1, NUM_CLASSES, H, W), lambda i: (i + K_SC, 0, 0, 0)),
    out_shape=jax.ShapeDtypeStruct((B, NUM_CLASSES, H, W), jnp.float32),
    input_output_aliases={1: 0},
)


def kernel(x):
    x3 = x.reshape(B, H, W)
    buf = _onehot_sc(x3)
    out = _tc_call(x3, buf)
    return out.reshape(B, 1, NUM_CLASSES, H, W)
